# baseline (speedup 1.0000x reference)
import functools

import jax
import jax.numpy as jnp
from jax import lax
from jax.experimental import pallas as pl
from jax.experimental.pallas import tpu as pltpu
from jax.experimental.pallas import tpu_sc as plsc

N = 10000
E = 320000
D = 128
HD = D // 2

NC = 2
NS = 16
C = 64
G = 16
NG = 20
NB = 8
NBH = NB // 2
NCH = NG * G
EPAD = NS * NCH * C
PADR = 112
NP = N + PADR
RPT = NP // NS
RPT_F = 632
RPT_LAST = N - (NS - 1) * RPT_F


def _sc_agg(hT, srcp, dstp, zeros):
    mesh = plsc.VectorSubcoreMesh(core_axis_name="c", subcore_axis_name="s")

    @functools.partial(
        pl.kernel,
        mesh=mesh,
        compiler_params=pltpu.CompilerParams(use_tc_tiling_on_sc=False),
        out_type=jax.ShapeDtypeStruct((NC, N, HD), jnp.float32),
        scratch_types=[
            pltpu.VMEM((3, G, C), jnp.int32),
            pltpu.VMEM((3, G, C), jnp.int32),
            pltpu.VMEM((NB, C, HD), jnp.float32),
            pltpu.VMEM_SHARED((NP, HD), jnp.float32),
            pltpu.VMEM_SHARED((NP, HD), jnp.float32),
        ] + [pltpu.SemaphoreType.DMA] * (2 * NB + 1),
    )
    def k(hT_hbm, src_hbm, dst_hbm, zero_hbm, out_hbm,
          src_v, dst_v, rows_v, hs_s, agg_s, *allsems):
        cid = lax.axis_index("c")
        sid = lax.axis_index("s")
        sems = allsems[:NB]
        ssems = allsems[NB:2 * NB]
        semi = allsems[2 * NB]

        pltpu.sync_copy(hT_hbm.at[cid, pl.ds(sid * RPT, RPT)],
                        hs_s.at[pl.ds(sid * RPT, RPT)])
        pltpu.sync_copy(zero_hbm.at[pl.ds(sid * RPT, RPT)],
                        agg_s.at[pl.ds(sid * RPT, RPT)])

        pltpu.sync_copy(src_hbm.at[sid, 0], src_v.at[0])
        pltpu.sync_copy(dst_hbm.at[sid, 0], dst_v.at[0])
        pltpu.async_copy(src_hbm.at[sid, 1], src_v.at[1], semi)
        pltpu.async_copy(dst_hbm.at[sid, 1], dst_v.at[1], semi)

        plsc.subcore_barrier()

        for b in range(NBH):
            pltpu.async_copy(hs_s.at[src_v.at[0, b]], rows_v.at[b],
                             sems[b])

        def outer(g, carry):
            @pl.when(g + 1 < NG)
            def _():
                pltpu.make_async_copy(src_hbm.at[sid, 0], src_v.at[0],
                                      semi).wait()
                pltpu.make_async_copy(dst_hbm.at[sid, 0], dst_v.at[0],
                                      semi).wait()

            def inner(jj2, c2):
                for b in range(NB):
                    jj = NB * jj2 + b
                    j = g * G + jj
                    rows = rows_v.at[b]
                    pltpu.make_async_copy(hs_s.at[src_v.at[0, 0]],
                                          rows, sems[b]).wait()
                    pltpu.async_copy(rows,
                                     agg_s.at[dst_v.at[g % 3, jj]],
                                     ssems[b], add=True)
                    bo = (b - NBH) % NB

                    @pl.when(j - NBH >= 0)
                    def _():
                        pltpu.make_async_copy(
                            rows_v.at[bo],
                            agg_s.at[dst_v.at[0, 0]],
                            ssems[bo]).wait()

                    jn = j + NBH
                    bn = (b + NBH) % NB

                    @pl.when(jn < NCH)
                    def _():
                        gslot = (jn // G) % 3
                        jjn = jn % G
                        pltpu.async_copy(
                            hs_s.at[src_v.at[gslot, jjn]],
                            rows_v.at[bn], sems[bn])
                return c2

            lax.fori_loop(0, G // NB, inner, 0)

            @pl.when(g + 2 < NG)
            def _():
                pltpu.async_copy(src_hbm.at[sid, g + 2],
                                 src_v.at[(g + 2) % 3], semi)
                pltpu.async_copy(dst_hbm.at[sid, g + 2],
                                 dst_v.at[(g + 2) % 3], semi)
            return carry

        lax.fori_loop(0, NG, outer, 0)

        for jtail in range(NCH - NBH, NCH):
            pltpu.make_async_copy(rows_v.at[jtail % NB],
                                  agg_s.at[dst_v.at[0, 0]],
                                  ssems[jtail % NB]).wait()

        plsc.subcore_barrier()

        @pl.when(sid < NS - 1)
        def _():
            pltpu.sync_copy(
                agg_s.at[pl.ds(sid * RPT_F, RPT_F)],
                out_hbm.at[cid, pl.ds(sid * RPT_F, RPT_F)])

        @pl.when(sid == NS - 1)
        def _():
            pltpu.sync_copy(
                agg_s.at[pl.ds((NS - 1) * RPT_F, RPT_LAST)],
                out_hbm.at[cid, pl.ds((NS - 1) * RPT_F, RPT_LAST)])

    return k(hT, srcp, dstp, zeros)


def _mlp_compute(h_ref, a_ref, w1_ref, b1_ref, w2_ref, b2_ref):
    a = a_ref[...]
    agg = jnp.concatenate([a[0], a[1]], axis=-1)
    z = h_ref[...] + agg
    z = jnp.maximum(
        jnp.dot(z, w1_ref[...], preferred_element_type=jnp.float32)
        + b1_ref[...], 0.0)
    return (jnp.dot(z, w2_ref[...], preferred_element_type=jnp.float32)
            + b2_ref[...])


def _mlp_body(h_ref, a_ref, w1_ref, b1_ref, w2_ref, b2_ref, o_ref):
    o_ref[...] = _mlp_compute(h_ref, a_ref, w1_ref, b1_ref, w2_ref, b2_ref)


def _mlp2_body(h_ref, a_ref, w1_ref, b1_ref, w2_ref, b2_ref, o_ref, o2_ref):
    hn = _mlp_compute(h_ref, a_ref, w1_ref, b1_ref, w2_ref, b2_ref)
    o_ref[...] = hn
    o2_ref[0] = hn[:, :HD]
    o2_ref[1] = hn[:, HD:]


def _tc_mlp(h, parts, W1, b1, W2, b2, want_ht):
    R = 1000
    in_specs = [
        pl.BlockSpec((R, D), lambda i: (i, 0)),
        pl.BlockSpec((NC, R, HD), lambda i: (0, i, 0)),
        pl.BlockSpec((D, D), lambda i: (0, 0)),
        pl.BlockSpec((1, D), lambda i: (0, 0)),
        pl.BlockSpec((D, D), lambda i: (0, 0)),
        pl.BlockSpec((1, D), lambda i: (0, 0)),
    ]
    if want_ht:
        return pl.pallas_call(
            _mlp2_body,
            grid=(N // R,),
            in_specs=in_specs,
            out_specs=(pl.BlockSpec((R, D), lambda i: (i, 0)),
                       pl.BlockSpec((2, R, HD), lambda i: (0, i, 0))),
            out_shape=(jax.ShapeDtypeStruct((N, D), jnp.float32),
                       jax.ShapeDtypeStruct((2, NP, HD), jnp.float32)),
        )(h, parts, W1, b1, W2, b2)
    return pl.pallas_call(
        _mlp_body,
        grid=(N // R,),
        in_specs=in_specs,
        out_specs=pl.BlockSpec((R, D), lambda i: (i, 0)),
        out_shape=jax.ShapeDtypeStruct((N, D), jnp.float32),
    )(h, parts, W1, b1, W2, b2)


def kernel(x, edge_index, batch,
           W1_0, b1_0, W2_0, b2_0,
           W1_1, b1_1, W2_1, b2_1,
           W1_2, b1_2, W2_2, b2_2):
    params = [(W1_0, b1_0, W2_0, b2_0),
              (W1_1, b1_1, W2_1, b2_1),
              (W1_2, b1_2, W2_2, b2_2)]
    src = edge_index[0]
    dst = edge_index[1]
    pad = EPAD - E
    srcp = jnp.concatenate(
        [src, jnp.zeros((pad,), jnp.int32)]).reshape(NS, NG, G, C)
    dstp = jnp.concatenate(
        [dst, jnp.full((pad,), N, jnp.int32)]).reshape(NS, NG, G, C)
    zeros = jnp.zeros((NP, HD), jnp.float32)

    h = x
    hT = jnp.pad(
        jnp.transpose(x.reshape(N, 2, HD), (1, 0, 2)),
        ((0, 0), (0, PADR), (0, 0)))
    for l, (W1, b1, W2, b2) in enumerate(params):
        parts = _sc_agg(hT, srcp, dstp, zeros)
        res = _tc_mlp(h, parts, W1, b1.reshape(1, D), W2, b2.reshape(1, D),
                      want_ht=(l < 2))
        if l < 2:
            h, hT = res
        else:
            h = res
    return h


# speedup vs baseline: 1.0131x; 1.0131x over previous
import functools

import jax
import jax.numpy as jnp
from jax import lax
from jax.experimental import pallas as pl
from jax.experimental.pallas import tpu as pltpu
from jax.experimental.pallas import tpu_sc as plsc

N = 10000
E = 320000
D = 128
HD = D // 2

NC = 2
NS = 16
C = 64
G = 16
NG = 20
NB = 8
NBH = NB // 2
NCH = NG * G
EPAD = NS * NCH * C
PADR = 112
NP = N + PADR
RPT = NP // NS
RPT_F = 632
RPT_LAST = N - (NS - 1) * RPT_F


def _sc_agg(hT, srcp, dstp):
    mesh = plsc.VectorSubcoreMesh(core_axis_name="c", subcore_axis_name="s")

    @functools.partial(
        pl.kernel,
        mesh=mesh,
        compiler_params=pltpu.CompilerParams(use_tc_tiling_on_sc=False),
        out_type=jax.ShapeDtypeStruct((NC, N, HD), jnp.float32),
        scratch_types=[
            pltpu.VMEM((3, G, C), jnp.int32),
            pltpu.VMEM((3, G, C), jnp.int32),
            pltpu.VMEM((NB, C, HD), jnp.float32),
            pltpu.VMEM_SHARED((NP, HD), jnp.float32),
            pltpu.VMEM_SHARED((NP, HD), jnp.float32),
        ] + [pltpu.SemaphoreType.DMA] * (2 * NB + 1),
    )
    def k(hT_hbm, src_hbm, dst_hbm, out_hbm,
          src_v, dst_v, rows_v, hs_s, agg_s, *allsems):
        cid = lax.axis_index("c")
        sid = lax.axis_index("s")
        sems = allsems[:NB]
        ssems = allsems[NB:2 * NB]
        semi = allsems[2 * NB]

        pltpu.sync_copy(hT_hbm.at[cid, pl.ds(sid * RPT, RPT)],
                        hs_s.at[pl.ds(sid * RPT, RPT)])
        pltpu.sync_copy(hT_hbm.at[cid, pl.ds(sid * RPT, RPT)],
                        agg_s.at[pl.ds(sid * RPT, RPT)])

        pltpu.sync_copy(src_hbm.at[sid, 0], src_v.at[0])
        pltpu.sync_copy(dst_hbm.at[sid, 0], dst_v.at[0])
        pltpu.async_copy(src_hbm.at[sid, 1], src_v.at[1], semi)
        pltpu.async_copy(dst_hbm.at[sid, 1], dst_v.at[1], semi)

        plsc.subcore_barrier()

        for b in range(NBH):
            pltpu.async_copy(hs_s.at[src_v.at[0, b]], rows_v.at[b],
                             sems[b])

        def outer(g, carry):
            @pl.when(g + 1 < NG)
            def _():
                pltpu.make_async_copy(src_hbm.at[sid, 0], src_v.at[0],
                                      semi).wait()
                pltpu.make_async_copy(dst_hbm.at[sid, 0], dst_v.at[0],
                                      semi).wait()

            def inner(jj2, c2):
                for b in range(NB):
                    jj = NB * jj2 + b
                    j = g * G + jj
                    rows = rows_v.at[b]
                    pltpu.make_async_copy(hs_s.at[src_v.at[0, 0]],
                                          rows, sems[b]).wait()
                    pltpu.async_copy(rows,
                                     agg_s.at[dst_v.at[g % 3, jj]],
                                     ssems[b], add=True)
                    bo = (b - NBH) % NB

                    @pl.when(j - NBH >= 0)
                    def _():
                        pltpu.make_async_copy(
                            rows_v.at[bo],
                            agg_s.at[dst_v.at[0, 0]],
                            ssems[bo]).wait()

                    jn = j + NBH
                    bn = (b + NBH) % NB

                    @pl.when(jn < NCH)
                    def _():
                        gslot = (jn // G) % 3
                        jjn = jn % G
                        pltpu.async_copy(
                            hs_s.at[src_v.at[gslot, jjn]],
                            rows_v.at[bn], sems[bn])
                return c2

            lax.fori_loop(0, G // NB, inner, 0)

            @pl.when(g + 2 < NG)
            def _():
                pltpu.async_copy(src_hbm.at[sid, g + 2],
                                 src_v.at[(g + 2) % 3], semi)
                pltpu.async_copy(dst_hbm.at[sid, g + 2],
                                 dst_v.at[(g + 2) % 3], semi)
            return carry

        lax.fori_loop(0, NG, outer, 0)

        for jtail in range(NCH - NBH, NCH):
            pltpu.make_async_copy(rows_v.at[jtail % NB],
                                  agg_s.at[dst_v.at[0, 0]],
                                  ssems[jtail % NB]).wait()

        plsc.subcore_barrier()

        @pl.when(sid < NS - 1)
        def _():
            pltpu.sync_copy(
                agg_s.at[pl.ds(sid * RPT_F, RPT_F)],
                out_hbm.at[cid, pl.ds(sid * RPT_F, RPT_F)])

        @pl.when(sid == NS - 1)
        def _():
            pltpu.sync_copy(
                agg_s.at[pl.ds((NS - 1) * RPT_F, RPT_LAST)],
                out_hbm.at[cid, pl.ds((NS - 1) * RPT_F, RPT_LAST)])

    return k(hT, srcp, dstp)


def _mlp_compute(a_ref, w1_ref, b1_ref, w2_ref, b2_ref):
    a = a_ref[...]
    z = jnp.concatenate([a[0], a[1]], axis=-1)
    z = jnp.maximum(
        jnp.dot(z, w1_ref[...], preferred_element_type=jnp.float32)
        + b1_ref[...], 0.0)
    return (jnp.dot(z, w2_ref[...], preferred_element_type=jnp.float32)
            + b2_ref[...])


def _mlp_body(a_ref, w1_ref, b1_ref, w2_ref, b2_ref, o_ref):
    o_ref[...] = _mlp_compute(a_ref, w1_ref, b1_ref, w2_ref, b2_ref)


def _mlp2_body(a_ref, w1_ref, b1_ref, w2_ref, b2_ref, o2_ref):
    hn = _mlp_compute(a_ref, w1_ref, b1_ref, w2_ref, b2_ref)
    o2_ref[0] = hn[:, :HD]
    o2_ref[1] = hn[:, HD:]


def _tc_mlp(parts, W1, b1, W2, b2, want_ht):
    R = 1000
    in_specs = [
        pl.BlockSpec((NC, R, HD), lambda i: (0, i, 0)),
        pl.BlockSpec((D, D), lambda i: (0, 0)),
        pl.BlockSpec((1, D), lambda i: (0, 0)),
        pl.BlockSpec((D, D), lambda i: (0, 0)),
        pl.BlockSpec((1, D), lambda i: (0, 0)),
    ]
    if want_ht:
        return pl.pallas_call(
            _mlp2_body,
            grid=(N // R,),
            in_specs=in_specs,
            out_specs=pl.BlockSpec((2, R, HD), lambda i: (0, i, 0)),
            out_shape=jax.ShapeDtypeStruct((2, NP, HD), jnp.float32),
        )(parts, W1, b1, W2, b2)
    return pl.pallas_call(
        _mlp_body,
        grid=(N // R,),
        in_specs=in_specs,
        out_specs=pl.BlockSpec((R, D), lambda i: (i, 0)),
        out_shape=jax.ShapeDtypeStruct((N, D), jnp.float32),
    )(parts, W1, b1, W2, b2)


def kernel(x, edge_index, batch,
           W1_0, b1_0, W2_0, b2_0,
           W1_1, b1_1, W2_1, b2_1,
           W1_2, b1_2, W2_2, b2_2):
    params = [(W1_0, b1_0, W2_0, b2_0),
              (W1_1, b1_1, W2_1, b2_1),
              (W1_2, b1_2, W2_2, b2_2)]
    src = edge_index[0]
    dst = edge_index[1]
    pad = EPAD - E
    srcp = jnp.concatenate(
        [src, jnp.zeros((pad,), jnp.int32)]).reshape(NS, NG, G, C)
    dstp = jnp.concatenate(
        [dst, jnp.full((pad,), N, jnp.int32)]).reshape(NS, NG, G, C)

    hT = jnp.pad(
        jnp.transpose(x.reshape(N, 2, HD), (1, 0, 2)),
        ((0, 0), (0, PADR), (0, 0)))
    for l, (W1, b1, W2, b2) in enumerate(params):
        parts = _sc_agg(hT, srcp, dstp)
        res = _tc_mlp(parts, W1, b1.reshape(1, D), W2, b2.reshape(1, D),
                      want_ht=(l < 2))
        if l < 2:
            hT = res
        else:
            h = res
    return h
